# Initial kernel scaffold; baseline (speedup 1.0000x reference)
import jax
import jax.numpy as jnp
from jax.experimental import pallas as pl


def kernel(node_features, edge_index, W1, b1, W2, b2, Wd1, bd1, Wd2, bd2):
    raise NotImplementedError("write your pallas kernel here")


# speedup vs baseline: 3.4778x; 3.4778x over previous
import jax
import jax.numpy as jnp
from jax import lax
from jax.experimental import pallas as pl
from jax.experimental.pallas import tpu as pltpu
from jax.experimental.pallas import tpu_sc as plsc

N_NODES = 10000
N_EDGES = 320000
D = 128

NC = 2
NS = 16
HALF = 5120
N_PAD = NC * HALF
ACC_ROWS = HALF + 128
CHUNK = 128
NCHUNK = 157
EDGES_PAD = NS * NCHUNK * CHUNK
DST_PAD = 10016
ZROWS = 8
WB_ROWS = 160


def _spmm_body(src_hbm, dst_hbm, x_hbm, out_hbm,
               idx_s, idx_d, rows, wb, zbuf, acc_sp, sem):
    c = lax.axis_index("c")
    s = lax.axis_index("s")

    zero16 = jnp.zeros((16,), jnp.float32)
    for r in range(ZROWS):
        for l in range(D // 16):
            zbuf[r, pl.ds(l * 16, 16)] = zero16

    def zero_step(k, _):
        pltpu.sync_copy(zbuf, acc_sp.at[pl.ds((s + k * NS) * ZROWS, ZROWS)])
        return 0
    lax.fori_loop(0, ACC_ROWS // ZROWS // NS, zero_step, 0)

    pltpu.sync_copy(src_hbm.at[s], idx_s)
    pltpu.sync_copy(dst_hbm.at[s], idx_d)

    lo = c * HALF
    def remap_step(j, _):
        for l in range(CHUNK // 16):
            d = idx_d[j, pl.ds(l * 16, 16)]
            local = d - lo
            bad = jnp.logical_or(local < 0, local >= HALF)
            idx_d[j, pl.ds(l * 16, 16)] = jnp.where(
                bad, HALF + (d & 127), local)
        return 0
    lax.fori_loop(0, NCHUNK, remap_step, 0)

    plsc.subcore_barrier()

    def edge_step(j, _):
        pltpu.async_copy(x_hbm.at[idx_s.at[j]], rows, sem).wait()
        pltpu.sync_copy(rows, acc_sp.at[idx_d.at[j]], add=True)
        return 0
    lax.fori_loop(0, NCHUNK, edge_step, 0)

    plsc.subcore_barrier()

    def wb_step(k, _):
        base = s * (HALF // NS) + k * WB_ROWS
        pltpu.sync_copy(acc_sp.at[pl.ds(base, WB_ROWS)], wb)
        pltpu.sync_copy(wb, out_hbm.at[pl.ds(c * HALF + base, WB_ROWS)])
        return 0
    lax.fori_loop(0, HALF // NS // WB_ROWS, wb_step, 0)


def _sc_spmm(x, src3, dst3):
    mesh = plsc.VectorSubcoreMesh(core_axis_name="c", subcore_axis_name="s")
    return pl.kernel(
        _spmm_body,
        out_type=jax.ShapeDtypeStruct((N_PAD, D), jnp.float32),
        mesh=mesh,
        scratch_types=[
            pltpu.VMEM((NCHUNK, CHUNK), jnp.int32),
            pltpu.VMEM((NCHUNK, CHUNK), jnp.int32),
            pltpu.VMEM((CHUNK, D), jnp.float32),
            pltpu.VMEM((WB_ROWS, D), jnp.float32),
            pltpu.VMEM((ZROWS, D), jnp.float32),
            pltpu.VMEM_SHARED((ACC_ROWS, D), jnp.float32),
            pltpu.SemaphoreType.DMA,
        ],
    )(src3, dst3, x)


ROW_BLK = 1000


def _layer_body(p_ref, w_ref, b_ref, out_ref):
    y = lax.dot_general(p_ref[...], w_ref[...], (((1,), (1,)), ((), ())),
                        preferred_element_type=jnp.float32)
    out_ref[...] = jnp.maximum(y + b_ref[...], 0.0)


def _tc_layer(p, w, b):
    grid = N_NODES // ROW_BLK
    return pl.pallas_call(
        _layer_body,
        grid=(grid,),
        in_specs=[
            pl.BlockSpec((ROW_BLK, D), lambda i: (i, 0)),
            pl.BlockSpec((D, D), lambda i: (0, 0)),
            pl.BlockSpec((1, D), lambda i: (0, 0)),
        ],
        out_specs=pl.BlockSpec((ROW_BLK, D), lambda i: (i, 0)),
        out_shape=jax.ShapeDtypeStruct((N_NODES, D), jnp.float32),
    )(p[:N_NODES], w, b.reshape(1, D))


def _head_body(h_ref, wd1_ref, bd1_ref, wd2_ref, bd2_ref,
               out_ref, acc_ref):
    i = pl.program_id(0)

    @pl.when(i == 0)
    def _():
        acc_ref[...] = jnp.zeros_like(acc_ref)

    acc_ref[...] += jnp.sum(h_ref[...], axis=0, keepdims=True)

    @pl.when(i == pl.num_programs(0) - 1)
    def _():
        emb = acc_ref[...] * (1.0 / N_NODES)
        d = lax.dot_general(emb, wd1_ref[...], (((1,), (1,)), ((), ())),
                            preferred_element_type=jnp.float32)
        d = jnp.maximum(d + bd1_ref[...], 0.0)
        z = lax.dot_general(d, wd2_ref[...], (((1,), (1,)), ((), ())),
                            preferred_element_type=jnp.float32)
        out_ref[...] = jax.nn.sigmoid(z + bd2_ref[...])


def _tc_head(h, wd1, bd1, wd2, bd2):
    grid = N_NODES // ROW_BLK
    full = lambda i: (0, 0)
    return pl.pallas_call(
        _head_body,
        grid=(grid,),
        in_specs=[
            pl.BlockSpec((ROW_BLK, D), lambda i: (i, 0)),
            pl.BlockSpec((D, D), full),
            pl.BlockSpec((1, D), full),
            pl.BlockSpec((D, D), full),
            pl.BlockSpec((1, D), full),
        ],
        out_specs=pl.BlockSpec((1, D), full),
        out_shape=jax.ShapeDtypeStruct((1, D), jnp.float32),
        scratch_shapes=[pltpu.VMEM((1, D), jnp.float32)],
    )(h, wd1, bd1.reshape(1, D), wd2, bd2.reshape(1, D))


def kernel(node_features, edge_index, W1, b1, W2, b2, Wd1, bd1, Wd2, bd2):
    npad = EDGES_PAD - N_EDGES
    src = jnp.concatenate(
        [edge_index[0].astype(jnp.int32),
         jnp.zeros((npad,), jnp.int32)]).reshape(NS, NCHUNK, CHUNK)
    dst = jnp.concatenate(
        [edge_index[1].astype(jnp.int32),
         jnp.full((npad,), DST_PAD, jnp.int32)]).reshape(NS, NCHUNK, CHUNK)

    p1 = _sc_spmm(node_features, src, dst)
    h1 = _tc_layer(p1, W1, b1)
    p2 = _sc_spmm(h1, src, dst)
    h2 = _tc_layer(p2, W2, b2)
    policy = _tc_head(h2, Wd1, bd1, Wd2, bd2)
    return policy.reshape(D)
